# Initial kernel scaffold; baseline (speedup 1.0000x reference)
import functools

import jax
import jax.numpy as jnp
import numpy as np
from jax import lax
from jax.experimental import pallas as pl

_CLUSTERS_N = 2
_ITERATIONS = 10
_NB = 8


@functools.lru_cache(maxsize=None)
def _init_indices(b):
    keys = jax.random.split(jax.random.key(42), b)
    perm = jax.vmap(lambda k: jax.random.permutation(k, 512)[:_CLUSTERS_N])(keys)
    idx = np.asarray(jax.device_get(perm)).astype(np.int32)
    out = np.zeros((b, 128), dtype=np.int32)
    out[:, :_CLUSTERS_N] = idx
    return out


def _fused_kernel(init_ref, feat_ref, out0_ref, out1_ref):
    feat = feat_ref[...]
    nb, h, w, c = feat.shape
    hw = h * w

    m1 = jnp.max(feat, axis=1)
    maxv = jnp.max(m1, axis=1)
    lin = (lax.broadcasted_iota(jnp.int32, feat.shape, 1) * w
           + lax.broadcasted_iota(jnp.int32, feat.shape, 2))
    hit = jnp.where(feat == maxv[:, None, None, :], lin, hw)
    i1 = jnp.min(hit, axis=1)
    idx = jnp.min(i1, axis=1)

    idx_f = idx.astype(jnp.float32)
    r = jnp.floor(idx_f / float(w))
    cc = idx_f - r * float(w)

    ch = lax.broadcasted_iota(jnp.int32, (nb, c), 1)
    i0v = init_ref[:, 0:1]
    i1v = init_ref[:, 1:2]
    sel0 = (ch == i0v)
    sel1 = (ch == i1v)
    cy0 = jnp.sum(jnp.where(sel0, r, 0.0), axis=1, keepdims=True)
    cx0 = jnp.sum(jnp.where(sel0, cc, 0.0), axis=1, keepdims=True)
    cy1 = jnp.sum(jnp.where(sel1, r, 0.0), axis=1, keepdims=True)
    cx1 = jnp.sum(jnp.where(sel1, cc, 0.0), axis=1, keepdims=True)

    sum_r = jnp.sum(r, axis=1, keepdims=True)
    sum_c = jnp.sum(cc, axis=1, keepdims=True)
    total = jnp.float32(c)

    assign1 = None
    for _ in range(_ITERATIONS + 1):
        d0 = (r - cy0) ** 2 + (cc - cx0) ** 2
        d1 = (r - cy1) ** 2 + (cc - cx1) ** 2
        assign1 = (d1 < d0).astype(jnp.float32)
        n1 = jnp.sum(assign1, axis=1, keepdims=True)
        n0g = jnp.maximum(total - n1, 1.0)
        n1g = jnp.maximum(n1, 1.0)
        s_r1 = jnp.sum(r * assign1, axis=1, keepdims=True)
        s_c1 = jnp.sum(cc * assign1, axis=1, keepdims=True)
        cy0 = (sum_r - s_r1) / n0g
        cx0 = (sum_c - s_c1) / n0g
        cy1 = s_r1 / n1g
        cx1 = s_c1 / n1g

    mask1 = assign1[:, None, None, :]
    o1 = feat * mask1
    out1_ref[...] = o1
    out0_ref[...] = feat - o1


def kernel(feature_batch):
    b, h, w, c = feature_batch.shape
    init = jnp.asarray(_init_indices(b))
    grid = b // _NB
    return pl.pallas_call(
        _fused_kernel,
        grid=(grid,),
        in_specs=[
            pl.BlockSpec((_NB, 128), lambda i: (i, 0)),
            pl.BlockSpec((_NB, h, w, c), lambda i: (i, 0, 0, 0)),
        ],
        out_specs=[
            pl.BlockSpec((_NB, h, w, c), lambda i: (i, 0, 0, 0)),
            pl.BlockSpec((_NB, h, w, c), lambda i: (i, 0, 0, 0)),
        ],
        out_shape=[
            jax.ShapeDtypeStruct((b, h, w, c), feature_batch.dtype),
            jax.ShapeDtypeStruct((b, h, w, c), feature_batch.dtype),
        ],
    )(init, feature_batch)


# speedup vs baseline: 1.8678x; 1.8678x over previous
import functools

import jax
import jax.numpy as jnp
import numpy as np
from jax import lax
from jax.experimental import pallas as pl

_CLUSTERS_N = 2
_ITERATIONS = 10
_NB = 8


@functools.lru_cache(maxsize=None)
def _init_indices(b):
    with jax.ensure_compile_time_eval():
        keys = jax.random.split(jax.random.key(42), b)
        perm = jax.vmap(lambda k: jax.random.permutation(k, 512)[:_CLUSTERS_N])(keys)
        idx = np.asarray(jax.device_get(perm)).astype(np.int32)
    out = np.zeros((b, 128), dtype=np.int32)
    out[:, :_CLUSTERS_N] = idx
    return out


def _fused_kernel(init_ref, feat_ref, out0_ref, out1_ref):
    feat = feat_ref[...]
    nb, h, w, c = feat.shape
    hw = h * w

    m1 = jnp.max(feat, axis=1)
    maxv = jnp.max(m1, axis=1)
    lin = (lax.broadcasted_iota(jnp.int32, feat.shape, 1) * w
           + lax.broadcasted_iota(jnp.int32, feat.shape, 2))
    hit = jnp.where(feat == maxv[:, None, None, :], lin, hw)
    i1 = jnp.min(hit, axis=1)
    idx = jnp.min(i1, axis=1)

    idx_f = idx.astype(jnp.float32)
    r = jnp.floor(idx_f / float(w))
    cc = idx_f - r * float(w)

    ch = lax.broadcasted_iota(jnp.int32, (nb, c), 1)
    i0v = init_ref[:, 0:1]
    i1v = init_ref[:, 1:2]
    sel0 = (ch == i0v)
    sel1 = (ch == i1v)
    cy0 = jnp.sum(jnp.where(sel0, r, 0.0), axis=1, keepdims=True)
    cx0 = jnp.sum(jnp.where(sel0, cc, 0.0), axis=1, keepdims=True)
    cy1 = jnp.sum(jnp.where(sel1, r, 0.0), axis=1, keepdims=True)
    cx1 = jnp.sum(jnp.where(sel1, cc, 0.0), axis=1, keepdims=True)

    sum_r = jnp.sum(r, axis=1, keepdims=True)
    sum_c = jnp.sum(cc, axis=1, keepdims=True)
    total = jnp.float32(c)

    assign1 = None
    for _ in range(_ITERATIONS + 1):
        d0 = (r - cy0) ** 2 + (cc - cx0) ** 2
        d1 = (r - cy1) ** 2 + (cc - cx1) ** 2
        assign1 = (d1 < d0).astype(jnp.float32)
        n1 = jnp.sum(assign1, axis=1, keepdims=True)
        n0g = jnp.maximum(total - n1, 1.0)
        n1g = jnp.maximum(n1, 1.0)
        s_r1 = jnp.sum(r * assign1, axis=1, keepdims=True)
        s_c1 = jnp.sum(cc * assign1, axis=1, keepdims=True)
        cy0 = (sum_r - s_r1) / n0g
        cx0 = (sum_c - s_c1) / n0g
        cy1 = s_r1 / n1g
        cx1 = s_c1 / n1g

    mask1 = assign1[:, None, None, :]
    o1 = feat * mask1
    out1_ref[...] = o1
    out0_ref[...] = feat - o1


def kernel(feature_batch):
    b, h, w, c = feature_batch.shape
    init = jnp.asarray(_init_indices(b))
    grid = b // _NB
    return pl.pallas_call(
        _fused_kernel,
        grid=(grid,),
        in_specs=[
            pl.BlockSpec((_NB, 128), lambda i: (i, 0)),
            pl.BlockSpec((_NB, h, w, c), lambda i: (i, 0, 0, 0)),
        ],
        out_specs=[
            pl.BlockSpec((_NB, h, w, c), lambda i: (i, 0, 0, 0)),
            pl.BlockSpec((_NB, h, w, c), lambda i: (i, 0, 0, 0)),
        ],
        out_shape=[
            jax.ShapeDtypeStruct((b, h, w, c), feature_batch.dtype),
            jax.ShapeDtypeStruct((b, h, w, c), feature_batch.dtype),
        ],
    )(init, feature_batch)


# speedup vs baseline: 1.8687x; 1.0005x over previous
import functools

import jax
import jax.numpy as jnp
import numpy as np
from jax import lax
from jax.experimental import pallas as pl

_CLUSTERS_N = 2
_ITERATIONS = 10
_NB = 8


def _init_perm(b):
    keys = jax.random.split(jax.random.key(42), b)
    return jax.vmap(lambda k: jax.random.permutation(k, 512)[:_CLUSTERS_N])(keys)


@functools.lru_cache(maxsize=None)
def _init_indices(b):
    with jax.ensure_compile_time_eval():
        idx = np.asarray(jax.device_get(_init_perm(b))).astype(np.int32)
    out = np.zeros((b, 128), dtype=np.int32)
    out[:, :_CLUSTERS_N] = idx
    return out


def _init_for(b):
    try:
        return jnp.asarray(_init_indices(b))
    except Exception:
        perm = _init_perm(b).astype(jnp.int32)
        return jnp.zeros((b, 128), jnp.int32).at[:, :_CLUSTERS_N].set(perm)


def _fused_kernel(init_ref, feat_ref, out0_ref, out1_ref):
    feat = feat_ref[...]
    nb, h, w, c = feat.shape
    hw = h * w

    m1 = jnp.max(feat, axis=1)
    maxv = jnp.max(m1, axis=1)
    lin = (lax.broadcasted_iota(jnp.int32, feat.shape, 1) * w
           + lax.broadcasted_iota(jnp.int32, feat.shape, 2))
    hit = jnp.where(feat == maxv[:, None, None, :], lin, hw)
    i1 = jnp.min(hit, axis=1)
    idx = jnp.min(i1, axis=1)

    idx_f = idx.astype(jnp.float32)
    r = jnp.floor(idx_f / float(w))
    cc = idx_f - r * float(w)

    ch = lax.broadcasted_iota(jnp.int32, (nb, c), 1)
    i0v = init_ref[:, 0:1]
    i1v = init_ref[:, 1:2]
    sel0 = (ch == i0v)
    sel1 = (ch == i1v)
    cy0 = jnp.sum(jnp.where(sel0, r, 0.0), axis=1, keepdims=True)
    cx0 = jnp.sum(jnp.where(sel0, cc, 0.0), axis=1, keepdims=True)
    cy1 = jnp.sum(jnp.where(sel1, r, 0.0), axis=1, keepdims=True)
    cx1 = jnp.sum(jnp.where(sel1, cc, 0.0), axis=1, keepdims=True)

    sum_r = jnp.sum(r, axis=1, keepdims=True)
    sum_c = jnp.sum(cc, axis=1, keepdims=True)
    total = jnp.float32(c)

    assign1 = None
    for _ in range(_ITERATIONS + 1):
        d0 = (r - cy0) ** 2 + (cc - cx0) ** 2
        d1 = (r - cy1) ** 2 + (cc - cx1) ** 2
        assign1 = (d1 < d0).astype(jnp.float32)
        n1 = jnp.sum(assign1, axis=1, keepdims=True)
        n0g = jnp.maximum(total - n1, 1.0)
        n1g = jnp.maximum(n1, 1.0)
        s_r1 = jnp.sum(r * assign1, axis=1, keepdims=True)
        s_c1 = jnp.sum(cc * assign1, axis=1, keepdims=True)
        cy0 = (sum_r - s_r1) / n0g
        cx0 = (sum_c - s_c1) / n0g
        cy1 = s_r1 / n1g
        cx1 = s_c1 / n1g

    mask1 = assign1[:, None, None, :]
    o1 = feat * mask1
    out1_ref[...] = o1
    out0_ref[...] = feat - o1


def kernel(feature_batch):
    b, h, w, c = feature_batch.shape
    init = _init_for(b)
    grid = b // _NB
    return pl.pallas_call(
        _fused_kernel,
        grid=(grid,),
        in_specs=[
            pl.BlockSpec((_NB, 128), lambda i: (i, 0)),
            pl.BlockSpec((_NB, h, w, c), lambda i: (i, 0, 0, 0)),
        ],
        out_specs=[
            pl.BlockSpec((_NB, h, w, c), lambda i: (i, 0, 0, 0)),
            pl.BlockSpec((_NB, h, w, c), lambda i: (i, 0, 0, 0)),
        ],
        out_shape=[
            jax.ShapeDtypeStruct((b, h, w, c), feature_batch.dtype),
            jax.ShapeDtypeStruct((b, h, w, c), feature_batch.dtype),
        ],
    )(init, feature_batch)
